# Initial kernel scaffold; baseline (speedup 1.0000x reference)
import jax
import jax.numpy as jnp
from jax.experimental import pallas as pl


def kernel(context_words, target_words, negative_words, emb, W, b):
    raise NotImplementedError("write your pallas kernel here")


# speedup vs baseline: 3.7014x; 3.7014x over previous
import functools

import jax
import jax.numpy as jnp
from jax import lax
from jax.experimental import pallas as pl
from jax.experimental.pallas import tpu as pltpu
from jax.experimental.pallas import tpu_sc as plsc

VOCAB = 1000000
EMB = 64
BATCH = 16384
CTX = 20
NEG = 20

NUM_CORES = 2
NUM_SUBCORES = 16
NUM_WORKERS = NUM_CORES * NUM_SUBCORES

GATHER_CHUNK = 128
NBUF = 4

CTX_ROWS_W = CTX * BATCH // NUM_WORKERS
TGT_ROWS_W = BATCH // NUM_WORKERS
NEG_ROWS_W = NEG * BATCH // NUM_WORKERS
IDX_BUF = CTX_ROWS_W


def _make_sc_gather():
    mesh = plsc.VectorSubcoreMesh(core_axis_name="c", subcore_axis_name="s")

    @functools.partial(
        pl.kernel,
        mesh=mesh,
        out_type=(
            jax.ShapeDtypeStruct((CTX * BATCH, EMB), jnp.float32),
            jax.ShapeDtypeStruct((BATCH, EMB), jnp.float32),
            jax.ShapeDtypeStruct((NEG * BATCH, EMB), jnp.float32),
        ),
        scratch_types=[
            pltpu.VMEM((IDX_BUF,), jnp.int32),
            pltpu.VMEM((NBUF, GATHER_CHUNK, EMB), jnp.float32),
            pltpu.SemaphoreType.DMA((NBUF,)),
            pltpu.SemaphoreType.DMA((NBUF,)),
        ],
        compiler_params=pltpu.CompilerParams(use_tc_tiling_on_sc=False),
    )
    def sc_gather(emb_hbm, idx_hbm, ctx_hbm, tgt_hbm, neg_hbm,
                  idx_v, rows_v, gsem, wsem):
        wid = lax.axis_index("s") * NUM_CORES + lax.axis_index("c")

        def run_region(idx_base, out_hbm, out_base, rows_w):
            pltpu.sync_copy(idx_hbm.at[pl.ds(idx_base, rows_w)], idx_v
                            if rows_w == IDX_BUF else idx_v.at[pl.ds(0, rows_w)])

            nchunks = rows_w // GATHER_CHUNK
            outer = nchunks // NBUF

            def start_gather(k, b):
                pltpu.async_copy(
                    emb_hbm.at[idx_v.at[pl.ds(k * GATHER_CHUNK, GATHER_CHUNK)]],
                    rows_v.at[b], gsem.at[b])

            def wait_gather(b):
                pltpu.make_async_copy(
                    out_hbm.at[pl.ds(0, GATHER_CHUNK)], rows_v.at[b],
                    gsem.at[b]).wait()

            def start_wb(k, b):
                pltpu.async_copy(
                    rows_v.at[b],
                    out_hbm.at[pl.ds(out_base + k * GATHER_CHUNK, GATHER_CHUNK)],
                    wsem.at[b])

            def wait_wb(k, b):
                pltpu.make_async_copy(
                    rows_v.at[b],
                    out_hbm.at[pl.ds(out_base + k * GATHER_CHUNK, GATHER_CHUNK)],
                    wsem.at[b]).wait()

            for b in range(NBUF):
                start_gather(b, b)

            def body(g, carry):
                k0 = g * NBUF
                for b in range(NBUF):
                    wait_gather(b)
                    start_wb(k0 + b, b)

                @pl.when(g + 1 < outer)
                def _():
                    for b in range(NBUF):
                        wait_wb(k0 + b, b)
                        start_gather(k0 + NBUF + b, b)

                return carry

            lax.fori_loop(0, outer, body, 0)
            for b in range(NBUF):
                wait_wb((outer - 1) * NBUF + b, b)

        run_region(wid * CTX_ROWS_W, ctx_hbm, wid * CTX_ROWS_W, CTX_ROWS_W)
        run_region(CTX * BATCH + wid * TGT_ROWS_W, tgt_hbm,
                   wid * TGT_ROWS_W, TGT_ROWS_W)
        run_region((CTX + 1) * BATCH + wid * NEG_ROWS_W, neg_hbm,
                   wid * NEG_ROWS_W, NEG_ROWS_W)

    return sc_gather


_sc_gather = _make_sc_gather()

BS_CTX = 512
BS_TGT = 1024
NB_B = 64


def _ctx_matmul_kernel(x_ref, w_ref, b_ref, o_ref):
    c = pl.program_id(1)
    xs = x_ref[...] * (1.0 / CTX)
    part = lax.dot_general(
        xs, w_ref[...], (((1,), (1,)), ((), ())),
        preferred_element_type=jnp.float32)

    @pl.when(c == 0)
    def _():
        o_ref[...] = (part + b_ref[...]).reshape(BS_CTX, 1, 2 * EMB)

    @pl.when(c != 0)
    def _():
        o_ref[...] += part.reshape(BS_CTX, 1, 2 * EMB)


def _tgt_matmul_kernel(x_ref, w_ref, b_ref, o_ref):
    o_ref[...] = lax.dot_general(
        x_ref[...], w_ref[...], (((1,), (1,)), ((), ())),
        preferred_element_type=jnp.float32) + b_ref[...]


def _neg_matmul_kernel(x_ref, w_ref, b_ref, o_ref):
    r = lax.dot_general(
        x_ref[...], w_ref[...], (((1,), (1,)), ((), ())),
        preferred_element_type=jnp.float32) + b_ref[...]
    o_ref[...] = r.reshape(NB_B, NEG, 2 * EMB)


def kernel(context_words, target_words, negative_words, emb, W, b):
    ctx_idx = context_words.astype(jnp.int32).T.reshape(-1)
    tgt_idx = target_words.astype(jnp.int32)
    neg_idx = negative_words.astype(jnp.int32).reshape(-1)
    all_idx = jnp.concatenate([ctx_idx, tgt_idx, neg_idx])

    ctx_stage, tgt_stage, neg_stage = _sc_gather(emb, all_idx)

    b2d = b.reshape(1, 2 * EMB)

    context_out = pl.pallas_call(
        _ctx_matmul_kernel,
        grid=(BATCH // BS_CTX, CTX),
        in_specs=[
            pl.BlockSpec((BS_CTX, EMB), lambda i, c: (c * (BATCH // BS_CTX) + i, 0)),
            pl.BlockSpec((2 * EMB, EMB), lambda i, c: (0, 0)),
            pl.BlockSpec((1, 2 * EMB), lambda i, c: (0, 0)),
        ],
        out_specs=pl.BlockSpec((BS_CTX, 1, 2 * EMB), lambda i, c: (i, 0, 0)),
        out_shape=jax.ShapeDtypeStruct((BATCH, 1, 2 * EMB), jnp.float32),
        compiler_params=pltpu.CompilerParams(
            dimension_semantics=("parallel", "arbitrary")),
    )(ctx_stage, W, b2d)

    target_out = pl.pallas_call(
        _tgt_matmul_kernel,
        grid=(BATCH // BS_TGT,),
        in_specs=[
            pl.BlockSpec((BS_TGT, EMB), lambda i: (i, 0)),
            pl.BlockSpec((2 * EMB, EMB), lambda i: (0, 0)),
            pl.BlockSpec((1, 2 * EMB), lambda i: (0, 0)),
        ],
        out_specs=pl.BlockSpec((BS_TGT, 2 * EMB), lambda i: (i, 0)),
        out_shape=jax.ShapeDtypeStruct((BATCH, 2 * EMB), jnp.float32),
        compiler_params=pltpu.CompilerParams(
            dimension_semantics=("arbitrary",)),
    )(tgt_stage, W, b2d)

    negative_out = pl.pallas_call(
        _neg_matmul_kernel,
        grid=(BATCH // NB_B,),
        in_specs=[
            pl.BlockSpec((NB_B * NEG, EMB), lambda i: (i, 0)),
            pl.BlockSpec((2 * EMB, EMB), lambda i: (0, 0)),
            pl.BlockSpec((1, 2 * EMB), lambda i: (0, 0)),
        ],
        out_specs=pl.BlockSpec((NB_B, NEG, 2 * EMB), lambda i: (i, 0, 0)),
        out_shape=jax.ShapeDtypeStruct((BATCH, NEG, 2 * EMB), jnp.float32),
        compiler_params=pltpu.CompilerParams(
            dimension_semantics=("arbitrary",)),
    )(neg_stage, W, b2d)

    return (context_out, negative_out, target_out)


# speedup vs baseline: 4.8348x; 1.3062x over previous
import functools

import jax
import jax.numpy as jnp
from jax import lax
from jax.experimental import pallas as pl
from jax.experimental.pallas import tpu as pltpu
from jax.experimental.pallas import tpu_sc as plsc

VOCAB = 1000000
EMB = 64
BATCH = 16384
CTX = 20
NEG = 20

NUM_CORES = 2
NUM_SUBCORES = 16
NUM_WORKERS = NUM_CORES * NUM_SUBCORES

B_W = BATCH // NUM_WORKERS
GRP = 8
CHUNK_1D = 128
NBUF = 4


def _make_sc_gather():
    mesh = plsc.VectorSubcoreMesh(core_axis_name="c", subcore_axis_name="s")

    @functools.partial(
        pl.kernel,
        mesh=mesh,
        out_type=(
            jax.ShapeDtypeStruct((BATCH, CTX, EMB), jnp.float32),
            jax.ShapeDtypeStruct((BATCH, EMB), jnp.float32),
            jax.ShapeDtypeStruct((BATCH, NEG, EMB), jnp.float32),
        ),
        scratch_types=[
            pltpu.VMEM((B_W, CTX), jnp.int32),
            pltpu.VMEM((B_W,), jnp.int32),
            pltpu.VMEM((NBUF, GRP, CTX, EMB), jnp.float32),
            pltpu.VMEM((NBUF, CHUNK_1D, EMB), jnp.float32),
            pltpu.SemaphoreType.DMA((NBUF,)),
            pltpu.SemaphoreType.DMA((NBUF,)),
        ],
        compiler_params=pltpu.CompilerParams(use_tc_tiling_on_sc=False),
    )
    def sc_gather(emb_hbm, ctx_idx_hbm, tgt_idx_hbm, neg_idx_hbm,
                  ctx_hbm, tgt_hbm, neg_hbm,
                  idx2_v, idx1_v, rows2_v, rows1_v, gsem, wsem):
        wid = lax.axis_index("s") * NUM_CORES + lax.axis_index("c")
        b0 = wid * B_W

        def run_region2(idx_hbm, out_hbm):
            ngroups = B_W // GRP
            outer = ngroups // NBUF
            pltpu.sync_copy(idx_hbm.at[pl.ds(b0, B_W)], idx2_v)

            def start_gathers(grp, b):
                for i in range(GRP):
                    pltpu.async_copy(
                        emb_hbm.at[idx2_v.at[grp * GRP + i]],
                        rows2_v.at[b, i], gsem.at[b])

            def wait_gathers(b):
                pltpu.make_async_copy(
                    out_hbm.at[pl.ds(0, GRP)], rows2_v.at[b],
                    gsem.at[b]).wait()

            def start_wb(grp, b):
                pltpu.async_copy(
                    rows2_v.at[b],
                    out_hbm.at[pl.ds(b0 + grp * GRP, GRP)], wsem.at[b])

            def wait_wb(grp, b):
                pltpu.make_async_copy(
                    rows2_v.at[b],
                    out_hbm.at[pl.ds(b0 + grp * GRP, GRP)], wsem.at[b]).wait()

            for b in range(NBUF):
                start_gathers(b, b)

            def body(g, carry):
                k0 = g * NBUF
                for b in range(NBUF):
                    wait_gathers(b)
                    start_wb(k0 + b, b)

                @pl.when(g + 1 < outer)
                def _():
                    for b in range(NBUF):
                        wait_wb(k0 + b, b)
                        start_gathers(k0 + NBUF + b, b)

                return carry

            lax.fori_loop(0, outer, body, 0)
            for b in range(NBUF):
                wait_wb((outer - 1) * NBUF + b, b)

        def run_region1(idx_hbm, out_hbm):
            pltpu.sync_copy(idx_hbm.at[pl.ds(b0, B_W)], idx1_v)
            nchunks = B_W // CHUNK_1D
            for b in range(nchunks):
                pltpu.async_copy(
                    emb_hbm.at[idx1_v.at[pl.ds(b * CHUNK_1D, CHUNK_1D)]],
                    rows1_v.at[b], gsem.at[b])
            for b in range(nchunks):
                pltpu.make_async_copy(
                    out_hbm.at[pl.ds(0, CHUNK_1D)], rows1_v.at[b],
                    gsem.at[b]).wait()
                pltpu.async_copy(
                    rows1_v.at[b],
                    out_hbm.at[pl.ds(b0 + b * CHUNK_1D, CHUNK_1D)], wsem.at[b])
            for b in range(nchunks):
                pltpu.make_async_copy(
                    rows1_v.at[b],
                    out_hbm.at[pl.ds(b0 + b * CHUNK_1D, CHUNK_1D)],
                    wsem.at[b]).wait()

        run_region2(ctx_idx_hbm, ctx_hbm)
        run_region1(tgt_idx_hbm, tgt_hbm)
        run_region2(neg_idx_hbm, neg_hbm)

    return sc_gather


_sc_gather = _make_sc_gather()

BS_CTX = 256
BS_TGT = 2048
NB_B = 256


def _ctx_matmul_kernel(x_ref, w_ref, b_ref, o_ref):
    xs = x_ref[...].sum(axis=1) * (1.0 / CTX)
    r = lax.dot_general(
        xs, w_ref[...], (((1,), (1,)), ((), ())),
        preferred_element_type=jnp.float32) + b_ref[...]
    o_ref[...] = r.reshape(BS_CTX, 1, 2 * EMB)


def _tgt_matmul_kernel(x_ref, w_ref, b_ref, o_ref):
    o_ref[...] = lax.dot_general(
        x_ref[...], w_ref[...], (((1,), (1,)), ((), ())),
        preferred_element_type=jnp.float32) + b_ref[...]


def _neg_matmul_kernel(x_ref, w_ref, b_ref, o_ref):
    x = x_ref[...].reshape(NB_B * NEG, EMB)
    r = lax.dot_general(
        x, w_ref[...], (((1,), (1,)), ((), ())),
        preferred_element_type=jnp.float32) + b_ref[...]
    o_ref[...] = r.reshape(NB_B, NEG, 2 * EMB)


def kernel(context_words, target_words, negative_words, emb, W, b):
    ctx_stage, tgt_stage, neg_stage = _sc_gather(
        emb, context_words.astype(jnp.int32), target_words.astype(jnp.int32),
        negative_words.astype(jnp.int32))

    b2d = b.reshape(1, 2 * EMB)

    context_out = pl.pallas_call(
        _ctx_matmul_kernel,
        grid=(BATCH // BS_CTX,),
        in_specs=[
            pl.BlockSpec((BS_CTX, CTX, EMB), lambda i: (i, 0, 0)),
            pl.BlockSpec((2 * EMB, EMB), lambda i: (0, 0)),
            pl.BlockSpec((1, 2 * EMB), lambda i: (0, 0)),
        ],
        out_specs=pl.BlockSpec((BS_CTX, 1, 2 * EMB), lambda i: (i, 0, 0)),
        out_shape=jax.ShapeDtypeStruct((BATCH, 1, 2 * EMB), jnp.float32),
        compiler_params=pltpu.CompilerParams(
            dimension_semantics=("arbitrary",)),
    )(ctx_stage, W, b2d)

    target_out = pl.pallas_call(
        _tgt_matmul_kernel,
        grid=(BATCH // BS_TGT,),
        in_specs=[
            pl.BlockSpec((BS_TGT, EMB), lambda i: (i, 0)),
            pl.BlockSpec((2 * EMB, EMB), lambda i: (0, 0)),
            pl.BlockSpec((1, 2 * EMB), lambda i: (0, 0)),
        ],
        out_specs=pl.BlockSpec((BS_TGT, 2 * EMB), lambda i: (i, 0)),
        out_shape=jax.ShapeDtypeStruct((BATCH, 2 * EMB), jnp.float32),
        compiler_params=pltpu.CompilerParams(
            dimension_semantics=("arbitrary",)),
    )(tgt_stage, W, b2d)

    negative_out = pl.pallas_call(
        _neg_matmul_kernel,
        grid=(BATCH // NB_B,),
        in_specs=[
            pl.BlockSpec((NB_B, NEG, EMB), lambda i: (i, 0, 0)),
            pl.BlockSpec((2 * EMB, EMB), lambda i: (0, 0)),
            pl.BlockSpec((1, 2 * EMB), lambda i: (0, 0)),
        ],
        out_specs=pl.BlockSpec((NB_B, NEG, 2 * EMB), lambda i: (i, 0, 0)),
        out_shape=jax.ShapeDtypeStruct((BATCH, NEG, 2 * EMB), jnp.float32),
        compiler_params=pltpu.CompilerParams(
            dimension_semantics=("arbitrary",)),
    )(neg_stage, W, b2d)

    return (context_out, negative_out, target_out)


# speedup vs baseline: 5.7229x; 1.1837x over previous
import functools

import jax
import jax.numpy as jnp
from jax import lax
from jax.experimental import pallas as pl
from jax.experimental.pallas import tpu as pltpu
from jax.experimental.pallas import tpu_sc as plsc

VOCAB = 1000000
EMB = 64
BATCH = 16384
CTX = 20
NEG = 20
PD = 2 * EMB

NUM_CORES = 2
NUM_SUBCORES = 16
NUM_WORKERS = NUM_CORES * NUM_SUBCORES

B_W = BATCH // NUM_WORKERS
GRP = 4
CHUNK_1D = 64
NBUF = 4


def _make_sc_gather():
    mesh = plsc.VectorSubcoreMesh(core_axis_name="c", subcore_axis_name="s")

    @functools.partial(
        pl.kernel,
        mesh=mesh,
        out_type=(
            jax.ShapeDtypeStruct((BATCH * CTX, PD), jnp.float32),
            jax.ShapeDtypeStruct((BATCH, PD), jnp.float32),
            jax.ShapeDtypeStruct((BATCH * NEG, PD), jnp.float32),
        ),
        scratch_types=[
            pltpu.VMEM((B_W, CTX), jnp.int32),
            pltpu.VMEM((B_W,), jnp.int32),
            pltpu.VMEM((NBUF, GRP * CTX, PD), jnp.float32),
            pltpu.SemaphoreType.DMA((NBUF,)),
            pltpu.SemaphoreType.DMA((NBUF,)),
        ],
    )
    def sc_gather(emb_hbm, ctx_idx_hbm, tgt_idx_hbm, neg_idx_hbm,
                  ctx_hbm, tgt_hbm, neg_hbm,
                  idx2_v, idx1_v, rows2_v, gsem, wsem):
        wid = lax.axis_index("s") * NUM_CORES + lax.axis_index("c")
        b0 = wid * B_W

        def run_region2(idx_hbm, out_hbm):
            ngroups = B_W // GRP
            outer = ngroups // NBUF
            pltpu.sync_copy(idx_hbm.at[pl.ds(b0, B_W)], idx2_v)

            def start_gathers(grp, b):
                for i in range(GRP):
                    pltpu.async_copy(
                        emb_hbm.at[idx2_v.at[grp * GRP + i]],
                        rows2_v.at[b].at[pl.ds(i * CTX, CTX)], gsem.at[b])

            def wait_gathers(b):
                pltpu.make_async_copy(
                    out_hbm.at[pl.ds(0, GRP * CTX)], rows2_v.at[b],
                    gsem.at[b]).wait()

            def start_wb(grp, b):
                pltpu.async_copy(
                    rows2_v.at[b],
                    out_hbm.at[pl.ds(b0 * CTX + grp * GRP * CTX, GRP * CTX)],
                    wsem.at[b])

            def wait_wb(grp, b):
                pltpu.make_async_copy(
                    rows2_v.at[b],
                    out_hbm.at[pl.ds(b0 * CTX + grp * GRP * CTX, GRP * CTX)],
                    wsem.at[b]).wait()

            for b in range(NBUF):
                start_gathers(b, b)

            def body(g, carry):
                k0 = g * NBUF
                for b in range(NBUF):
                    wait_gathers(b)
                    start_wb(k0 + b, b)

                @pl.when(g + 1 < outer)
                def _():
                    for b in range(NBUF):
                        wait_wb(k0 + b, b)
                        start_gathers(k0 + NBUF + b, b)

                return carry

            lax.fori_loop(0, outer, body, 0)
            for b in range(NBUF):
                wait_wb((outer - 1) * NBUF + b, b)

        def run_region1(idx_hbm, out_hbm):
            pltpu.sync_copy(idx_hbm.at[pl.ds(b0, B_W)], idx1_v)
            nchunks = B_W // CHUNK_1D

            def tgt_rows(b):
                return rows2_v.at[b].at[pl.ds(0, CHUNK_1D)]

            def tgt_gather(c, b):
                pltpu.async_copy(
                    emb_hbm.at[idx1_v.at[pl.ds(c * CHUNK_1D, CHUNK_1D)]],
                    tgt_rows(b), gsem.at[b])

            def tgt_wb(c, b, sem):
                return pltpu.make_async_copy(
                    tgt_rows(b),
                    out_hbm.at[pl.ds(b0 + c * CHUNK_1D, CHUNK_1D)], sem)

            for b in range(NBUF):
                tgt_gather(b, b)
            for b in range(NBUF):
                pltpu.make_async_copy(
                    out_hbm.at[pl.ds(0, CHUNK_1D)], tgt_rows(b),
                    gsem.at[b]).wait()
                tgt_wb(b, b, wsem.at[b]).start()
            for b in range(NBUF):
                tgt_wb(b, b, wsem.at[b]).wait()
                tgt_gather(NBUF + b, b)
            for b in range(NBUF):
                pltpu.make_async_copy(
                    out_hbm.at[pl.ds(0, CHUNK_1D)], tgt_rows(b),
                    gsem.at[b]).wait()
                tgt_wb(NBUF + b, b, wsem.at[b]).start()
            for b in range(NBUF):
                tgt_wb(NBUF + b, b, wsem.at[b]).wait()

        run_region2(ctx_idx_hbm, ctx_hbm)
        run_region1(tgt_idx_hbm, tgt_hbm)
        run_region2(neg_idx_hbm, neg_hbm)

    return sc_gather


_sc_gather = _make_sc_gather()

BS_CTX = 256
BS_TGT = 2048
NB_B = 128


def _ctx_matmul_kernel(x_ref, w_ref, b_ref, o_ref):
    x = x_ref[...].reshape(BS_CTX, CTX, PD)[:, :, :EMB]
    xs = x.sum(axis=1) * (1.0 / CTX)
    r = lax.dot_general(
        xs, w_ref[...], (((1,), (1,)), ((), ())),
        preferred_element_type=jnp.float32) + b_ref[...]
    o_ref[...] = r.reshape(BS_CTX, 1, 2 * EMB)


def _tgt_matmul_kernel(x_ref, w_ref, b_ref, o_ref):
    o_ref[...] = lax.dot_general(
        x_ref[:, :EMB], w_ref[...], (((1,), (1,)), ((), ())),
        preferred_element_type=jnp.float32) + b_ref[...]


def _neg_matmul_kernel(x_ref, w_ref, b_ref, o_ref):
    r = lax.dot_general(
        x_ref[:, :EMB], w_ref[...], (((1,), (1,)), ((), ())),
        preferred_element_type=jnp.float32) + b_ref[...]
    o_ref[...] = r.reshape(NB_B, NEG, 2 * EMB)


def kernel(context_words, target_words, negative_words, emb, W, b):
    emb_pad = jnp.pad(emb, ((0, 0), (0, PD - EMB)))

    ctx_stage, tgt_stage, neg_stage = _sc_gather(
        emb_pad, context_words.astype(jnp.int32),
        target_words.astype(jnp.int32), negative_words.astype(jnp.int32))

    b2d = b.reshape(1, 2 * EMB)

    context_out = pl.pallas_call(
        _ctx_matmul_kernel,
        grid=(BATCH // BS_CTX,),
        in_specs=[
            pl.BlockSpec((BS_CTX * CTX, PD), lambda i: (i, 0)),
            pl.BlockSpec((2 * EMB, EMB), lambda i: (0, 0)),
            pl.BlockSpec((1, 2 * EMB), lambda i: (0, 0)),
        ],
        out_specs=pl.BlockSpec((BS_CTX, 1, 2 * EMB), lambda i: (i, 0, 0)),
        out_shape=jax.ShapeDtypeStruct((BATCH, 1, 2 * EMB), jnp.float32),
        compiler_params=pltpu.CompilerParams(
            dimension_semantics=("arbitrary",)),
    )(ctx_stage, W, b2d)

    target_out = pl.pallas_call(
        _tgt_matmul_kernel,
        grid=(BATCH // BS_TGT,),
        in_specs=[
            pl.BlockSpec((BS_TGT, PD), lambda i: (i, 0)),
            pl.BlockSpec((2 * EMB, EMB), lambda i: (0, 0)),
            pl.BlockSpec((1, 2 * EMB), lambda i: (0, 0)),
        ],
        out_specs=pl.BlockSpec((BS_TGT, 2 * EMB), lambda i: (i, 0)),
        out_shape=jax.ShapeDtypeStruct((BATCH, 2 * EMB), jnp.float32),
        compiler_params=pltpu.CompilerParams(
            dimension_semantics=("arbitrary",)),
    )(tgt_stage, W, b2d)

    negative_out = pl.pallas_call(
        _neg_matmul_kernel,
        grid=(BATCH // NB_B,),
        in_specs=[
            pl.BlockSpec((NB_B * NEG, PD), lambda i: (i, 0)),
            pl.BlockSpec((2 * EMB, EMB), lambda i: (0, 0)),
            pl.BlockSpec((1, 2 * EMB), lambda i: (0, 0)),
        ],
        out_specs=pl.BlockSpec((NB_B, NEG, 2 * EMB), lambda i: (i, 0, 0)),
        out_shape=jax.ShapeDtypeStruct((BATCH, NEG, 2 * EMB), jnp.float32),
        compiler_params=pltpu.CompilerParams(
            dimension_semantics=("arbitrary",)),
    )(neg_stage, W, b2d)

    return (context_out, negative_out, target_out)


# speedup vs baseline: 6.5952x; 1.1524x over previous
import functools

import jax
import jax.numpy as jnp
from jax import lax
from jax.experimental import pallas as pl
from jax.experimental.pallas import tpu as pltpu
from jax.experimental.pallas import tpu_sc as plsc

VOCAB = 1000000
EMB = 64
BATCH = 16384
CTX = 20
NEG = 20
PD = 2 * EMB

NUM_CORES = 2
NUM_SUBCORES = 16
NUM_WORKERS = NUM_CORES * NUM_SUBCORES

B_W = BATCH // NUM_WORKERS
GRP = 4
CHUNK_1D = 64
NBUF = 4
POOL_E = 128


def _worker_base():
    wid = lax.axis_index("s") * NUM_CORES + lax.axis_index("c")
    return wid * B_W


def _run_gather_region(emb_hbm, idx_hbm, out_hbm, b0, idx2_v, rows2_v,
                       gsem, wsem):
    ngroups = B_W // GRP
    outer = ngroups // NBUF
    pltpu.sync_copy(idx_hbm.at[pl.ds(b0, B_W)], idx2_v)

    def start_gathers(grp, b):
        for i in range(GRP):
            pltpu.async_copy(
                emb_hbm.at[idx2_v.at[grp * GRP + i]],
                rows2_v.at[b].at[pl.ds(i * CTX, CTX)], gsem.at[b])

    def wait_gathers(b):
        pltpu.make_async_copy(
            out_hbm.at[pl.ds(0, GRP * CTX)], rows2_v.at[b],
            gsem.at[b]).wait()

    def start_wb(grp, b):
        pltpu.async_copy(
            rows2_v.at[b],
            out_hbm.at[pl.ds(b0 * CTX + grp * GRP * CTX, GRP * CTX)],
            wsem.at[b])

    def wait_wb(grp, b):
        pltpu.make_async_copy(
            rows2_v.at[b],
            out_hbm.at[pl.ds(b0 * CTX + grp * GRP * CTX, GRP * CTX)],
            wsem.at[b]).wait()

    for b in range(NBUF):
        start_gathers(b, b)

    def body(g, carry):
        k0 = g * NBUF
        for b in range(NBUF):
            wait_gathers(b)
            start_wb(k0 + b, b)

        @pl.when(g + 1 < outer)
        def _():
            for b in range(NBUF):
                wait_wb(k0 + b, b)
                start_gathers(k0 + NBUF + b, b)

        return carry

    lax.fori_loop(0, outer, body, 0)
    for b in range(NBUF):
        wait_wb((outer - 1) * NBUF + b, b)


def _run_pooled_region(emb_hbm, idx_hbm, out_hbm, b0, idx2_v, rows2_v,
                       pool_v, gsem, wsem):
    ngroups = B_W // GRP
    outer = ngroups // NBUF
    pltpu.sync_copy(idx_hbm.at[pl.ds(b0, B_W)], idx2_v)

    def start_gathers(grp, b):
        for i in range(GRP):
            pltpu.async_copy(
                emb_hbm.at[idx2_v.at[grp * GRP + i]],
                rows2_v.at[b].at[pl.ds(i * CTX, CTX)], gsem.at[b])

    def wait_gathers(b):
        pltpu.make_async_copy(
            out_hbm.at[pl.ds(0, GRP * CTX)], rows2_v.at[b],
            gsem.at[b]).wait()

    def pool_group(local_grp, b):
        for i in range(GRP):
            base = i * CTX
            for c in range(EMB // 16):
                sl = pl.ds(c * 16, 16)
                acc = rows2_v[b, base, sl]
                for r in range(1, CTX):
                    acc = acc + rows2_v[b, base + r, sl]
                pool_v[local_grp * GRP + i, sl] = acc

    for b in range(NBUF):
        start_gathers(b, b)

    g_per_chunk = POOL_E // (NBUF * GRP)

    def body(g, carry):
        k0 = g * NBUF
        gm = lax.rem(g, g_per_chunk)
        for b in range(NBUF):
            wait_gathers(b)
            pool_group(gm * NBUF + b, b)

            @pl.when(g + 1 < outer)
            def _():
                start_gathers(k0 + NBUF + b, b)

        @pl.when(gm == g_per_chunk - 1)
        def _():
            chunk = lax.div(g, g_per_chunk)
            dst = out_hbm.at[pl.ds(b0 + chunk * POOL_E, POOL_E)]
            pltpu.async_copy(pool_v, dst, wsem.at[0])
            pltpu.make_async_copy(pool_v, dst, wsem.at[0]).wait()

        return carry

    lax.fori_loop(0, outer, body, 0)


def _run_1d_region(emb_hbm, idx_hbm, out_hbm, b0, idx1_v, rows2_v,
                   gsem, wsem):
    pltpu.sync_copy(idx_hbm.at[pl.ds(b0, B_W)], idx1_v)

    def tgt_rows(b):
        return rows2_v.at[b].at[pl.ds(0, CHUNK_1D)]

    def tgt_gather(c, b):
        pltpu.async_copy(
            emb_hbm.at[idx1_v.at[pl.ds(c * CHUNK_1D, CHUNK_1D)]],
            tgt_rows(b), gsem.at[b])

    def tgt_wb(c, b, sem):
        return pltpu.make_async_copy(
            tgt_rows(b),
            out_hbm.at[pl.ds(b0 + c * CHUNK_1D, CHUNK_1D)], sem)

    for b in range(NBUF):
        tgt_gather(b, b)
    for b in range(NBUF):
        pltpu.make_async_copy(
            out_hbm.at[pl.ds(0, CHUNK_1D)], tgt_rows(b), gsem.at[b]).wait()
        tgt_wb(b, b, wsem.at[b]).start()
    for b in range(NBUF):
        tgt_wb(b, b, wsem.at[b]).wait()
        tgt_gather(NBUF + b, b)
    for b in range(NBUF):
        pltpu.make_async_copy(
            out_hbm.at[pl.ds(0, CHUNK_1D)], tgt_rows(b), gsem.at[b]).wait()
        tgt_wb(NBUF + b, b, wsem.at[b]).start()
    for b in range(NBUF):
        tgt_wb(NBUF + b, b, wsem.at[b]).wait()


def _make_sc_neg():
    mesh = plsc.VectorSubcoreMesh(core_axis_name="c", subcore_axis_name="s")

    @functools.partial(
        pl.kernel,
        mesh=mesh,
        out_type=jax.ShapeDtypeStruct((BATCH * NEG, PD), jnp.float32),
        scratch_types=[
            pltpu.VMEM((B_W, CTX), jnp.int32),
            pltpu.VMEM((NBUF, GRP * CTX, PD), jnp.float32),
            pltpu.SemaphoreType.DMA((NBUF,)),
            pltpu.SemaphoreType.DMA((NBUF,)),
        ],
    )
    def sc_neg(emb_hbm, neg_idx_hbm, neg_hbm, idx2_v, rows2_v, gsem, wsem):
        b0 = _worker_base()
        _run_gather_region(emb_hbm, neg_idx_hbm, neg_hbm, b0,
                           idx2_v, rows2_v, gsem, wsem)

    return sc_neg


def _make_sc_ctx_tgt():
    mesh = plsc.VectorSubcoreMesh(core_axis_name="c", subcore_axis_name="s")

    @functools.partial(
        pl.kernel,
        mesh=mesh,
        out_type=(
            jax.ShapeDtypeStruct((BATCH, PD), jnp.float32),
            jax.ShapeDtypeStruct((BATCH, PD), jnp.float32),
        ),
        scratch_types=[
            pltpu.VMEM((B_W, CTX), jnp.int32),
            pltpu.VMEM((B_W,), jnp.int32),
            pltpu.VMEM((NBUF, GRP * CTX, PD), jnp.float32),
            pltpu.VMEM((POOL_E, PD), jnp.float32),
            pltpu.SemaphoreType.DMA((NBUF,)),
            pltpu.SemaphoreType.DMA((NBUF,)),
        ],
    )
    def sc_ctx_tgt(emb_hbm, ctx_idx_hbm, tgt_idx_hbm, ctx_hbm, tgt_hbm,
                   idx2_v, idx1_v, rows2_v, pool_v, gsem, wsem):
        b0 = _worker_base()
        _run_pooled_region(emb_hbm, ctx_idx_hbm, ctx_hbm, b0,
                           idx2_v, rows2_v, pool_v, gsem, wsem)
        _run_1d_region(emb_hbm, tgt_idx_hbm, tgt_hbm, b0,
                       idx1_v, rows2_v, gsem, wsem)

    return sc_ctx_tgt


_sc_neg = _make_sc_neg()
_sc_ctx_tgt = _make_sc_ctx_tgt()

BS = 256


def _neg_matmul_kernel(x_ref, w_ref, b_ref, o_ref):
    r = lax.dot_general(
        x_ref[:, :EMB], w_ref[...], (((1,), (1,)), ((), ())),
        preferred_element_type=jnp.float32) + b_ref[...]
    o_ref[...] = r.reshape(BS, NEG, 2 * EMB)


def _ctx_tgt_matmul_kernel(ctx_ref, tgt_ref, w_ref, b_ref, ctx_o, tgt_o):
    w = w_ref[...]
    bb = b_ref[...]
    xcs = ctx_ref[:, :EMB] * (1.0 / CTX)
    rc = lax.dot_general(
        xcs, w, (((1,), (1,)), ((), ())),
        preferred_element_type=jnp.float32) + bb
    ctx_o[...] = rc.reshape(BS, 1, 2 * EMB)
    tgt_o[...] = lax.dot_general(
        tgt_ref[:, :EMB], w, (((1,), (1,)), ((), ())),
        preferred_element_type=jnp.float32) + bb


def kernel(context_words, target_words, negative_words, emb, W, b):
    emb_pad = jnp.pad(emb, ((0, 0), (0, PD - EMB)))

    neg_stage = _sc_neg(emb_pad, negative_words.astype(jnp.int32))
    ctx_stage, tgt_stage = _sc_ctx_tgt(
        emb_pad, context_words.astype(jnp.int32),
        target_words.astype(jnp.int32))

    b2d = b.reshape(1, 2 * EMB)

    negative_out = pl.pallas_call(
        _neg_matmul_kernel,
        grid=(BATCH // BS,),
        in_specs=[
            pl.BlockSpec((BS * NEG, PD), lambda i: (i, 0)),
            pl.BlockSpec((2 * EMB, EMB), lambda i: (0, 0)),
            pl.BlockSpec((1, 2 * EMB), lambda i: (0, 0)),
        ],
        out_specs=pl.BlockSpec((BS, NEG, 2 * EMB), lambda i: (i, 0, 0)),
        out_shape=jax.ShapeDtypeStruct((BATCH, NEG, 2 * EMB), jnp.float32),
        compiler_params=pltpu.CompilerParams(
            dimension_semantics=("arbitrary",)),
    )(neg_stage, W, b2d)

    context_out, target_out = pl.pallas_call(
        _ctx_tgt_matmul_kernel,
        grid=(BATCH // BS,),
        in_specs=[
            pl.BlockSpec((BS, PD), lambda i: (i, 0)),
            pl.BlockSpec((BS, PD), lambda i: (i, 0)),
            pl.BlockSpec((2 * EMB, EMB), lambda i: (0, 0)),
            pl.BlockSpec((1, 2 * EMB), lambda i: (0, 0)),
        ],
        out_specs=[
            pl.BlockSpec((BS, 1, 2 * EMB), lambda i: (i, 0, 0)),
            pl.BlockSpec((BS, 2 * EMB), lambda i: (i, 0)),
        ],
        out_shape=(
            jax.ShapeDtypeStruct((BATCH, 1, 2 * EMB), jnp.float32),
            jax.ShapeDtypeStruct((BATCH, 2 * EMB), jnp.float32),
        ),
        compiler_params=pltpu.CompilerParams(
            dimension_semantics=("arbitrary",)),
    )(ctx_stage, tgt_stage, W, b2d)

    return (context_out, negative_out, target_out)
